# Initial kernel scaffold; baseline (speedup 1.0000x reference)
import jax
import jax.numpy as jnp
from jax.experimental import pallas as pl


def kernel(x, edge_index, W1, b1, gw1, W2, b2, gw2, W3, b3):
    raise NotImplementedError("write your pallas kernel here")


# speedup vs baseline: 18.7650x; 18.7650x over previous
import functools

import jax
import jax.numpy as jnp
from jax import lax
from jax.experimental import pallas as pl
from jax.experimental.pallas import tpu as pltpu
from jax.experimental.pallas import tpu_sc as plsc

_NC = 2
_NS = 16
_NW = _NC * _NS
_L = 16


def _project_body(x_ref, w1_ref, b1r_ref, gw1_ref, w2_ref, gw2_ref, w3_ref,
                  u_ref):
    g = (w1_ref[...] * gw1_ref[...]) @ w2_ref[...]
    wc = (g * gw2_ref[...]) @ w3_ref[...]
    c1 = (((b1r_ref[...] * gw1_ref[...]) @ w2_ref[...])
          * gw2_ref[...]) @ w3_ref[...]
    u_ref[...] = x_ref[...] @ wc + c1


def _combine_body(p_ref, o_ref):
    i = pl.program_id(0)

    @pl.when(i == 0)
    def _():
        o_ref[...] = jnp.zeros_like(o_ref)

    o_ref[...] += p_ref[...]


@functools.lru_cache(maxsize=None)
def _make_scatter_pass(n, e, fo):
    nf = n * fo
    nfp = -(-nf // 1024) * 1024
    ept = e // _NW
    assert ept * _NW == e and ept % _L == 0 and nf % _L == 0

    mesh = plsc.VectorSubcoreMesh(core_axis_name="c", subcore_axis_name="s")

    @functools.partial(
        pl.kernel,
        out_type=jax.ShapeDtypeStruct((_NW * nfp,), jnp.float32),
        mesh=mesh,
        compiler_params=pltpu.CompilerParams(needs_layout_passes=False),
        scratch_types=[
            pltpu.VMEM((nf,), jnp.float32),
            pltpu.VMEM((nfp,), jnp.float32),
            pltpu.VMEM((ept,), jnp.int32),
            pltpu.VMEM((ept,), jnp.int32),
            pltpu.SemaphoreType.DMA,
        ],
    )
    def scatter_pass(u_hbm, row_hbm, col_hbm, out_hbm,
                     u_v, v_v, row_v, col_v, sem):
        c = lax.axis_index("c")
        s = lax.axis_index("s")
        wid = s * _NC + c
        base = wid * ept
        cp_r = pltpu.async_copy(row_hbm.at[pl.ds(base, ept)], row_v, sem)
        cp_c = pltpu.async_copy(col_hbm.at[pl.ds(base, ept)], col_v, sem)
        cp_u = pltpu.async_copy(u_hbm, u_v, sem)

        zero = jnp.zeros((_L,), jnp.float32)

        def zbody(i, carry):
            v_v[pl.ds(i * _L, _L)] = zero
            return carry

        lax.fori_loop(0, nfp // _L, zbody, 0)
        cp_r.wait()
        cp_c.wait()
        cp_u.wait()

        def body(i, carry):
            cv = col_v[pl.ds(i * _L, _L)]
            rv = row_v[pl.ds(i * _L, _L)]
            cb = cv * fo
            rb = rv * fo
            for f in range(fo):
                gi = cb + f if f else cb
                si = rb + f if f else rb
                g = plsc.load_gather(u_v, [gi])
                plsc.addupdate_scatter(v_v, [si], g)
            return carry

        lax.fori_loop(0, ept // _L, body, 0)
        pltpu.sync_copy(v_v, out_hbm.at[pl.ds(wid * nfp, nfp)])

    return scatter_pass


def _combine(partials, nfp):
    return pl.pallas_call(
        _combine_body,
        grid=(_NW,),
        in_specs=[pl.BlockSpec((nfp,), lambda i: (i,))],
        out_specs=pl.BlockSpec((nfp,), lambda i: (0,)),
        out_shape=jax.ShapeDtypeStruct((nfp,), jnp.float32),
    )(partials)


def kernel(x, edge_index, W1, b1, gw1, W2, b2, gw2, W3, b3):
    n, _ = x.shape
    e = edge_index.shape[1]
    fo = W3.shape[1]
    nf = n * fo
    nfp = -(-nf // 1024) * 1024

    row = edge_index[0]
    col = edge_index[1]
    b1r = b1.reshape(1, -1)

    u2d = pl.pallas_call(
        _project_body,
        out_shape=jax.ShapeDtypeStruct((n, fo), jnp.float32),
    )(x, W1, b1r, gw1, W2, gw2, W3)
    u = u2d.reshape(nf)

    scatter = _make_scatter_pass(n, e, fo)

    p_part = scatter(u, row, col)
    v = _combine(p_part, nfp)[:nf]
    c2 = (b2 * gw2[0]) @ W3
    v = v + jnp.tile(c2, n)

    q_part = scatter(v, row, col)
    out_flat = _combine(q_part, nfp)[:nf]
    return out_flat.reshape(n, fo) + b3[None, :]


# speedup vs baseline: 20.0666x; 1.0694x over previous
import functools

import jax
import jax.numpy as jnp
from jax import lax
from jax.experimental import pallas as pl
from jax.experimental.pallas import tpu as pltpu
from jax.experimental.pallas import tpu_sc as plsc

_NC = 2
_NS = 16
_NW = _NC * _NS
_L = 16


def _project_body(x_ref, w1_ref, b1r_ref, gw1_ref, w2_ref, gw2_ref, w3_ref,
                  u_ref):
    g = (w1_ref[...] * gw1_ref[...]) @ w2_ref[...]
    wc = (g * gw2_ref[...]) @ w3_ref[...]
    c1 = (((b1r_ref[...] * gw1_ref[...]) @ w2_ref[...])
          * gw2_ref[...]) @ w3_ref[...]
    u_ref[...] = x_ref[...] @ wc + c1


def _combine_body(p_ref, o_ref):
    i = pl.program_id(0)

    @pl.when(i == 0)
    def _():
        o_ref[...] = jnp.zeros_like(o_ref)

    o_ref[...] += p_ref[...]


@functools.lru_cache(maxsize=None)
def _make_scatter_pass(n, e, fo):
    nf = n * fo
    nfp = -(-nf // 1024) * 1024
    ept = e // _NW
    assert ept * _NW == e and ept % _L == 0 and nf % _L == 0

    mesh = plsc.VectorSubcoreMesh(core_axis_name="c", subcore_axis_name="s")

    @functools.partial(
        pl.kernel,
        out_type=jax.ShapeDtypeStruct((_NW * nfp,), jnp.float32),
        mesh=mesh,
        compiler_params=pltpu.CompilerParams(needs_layout_passes=False),
        scratch_types=[
            pltpu.VMEM((nf,), jnp.float32),
            pltpu.VMEM((nfp,), jnp.float32),
            pltpu.VMEM((ept,), jnp.int32),
            pltpu.VMEM((ept,), jnp.int32),
            pltpu.SemaphoreType.DMA,
        ],
    )
    def scatter_pass(u_hbm, row_hbm, col_hbm, out_hbm,
                     u_v, v_v, row_v, col_v, sem):
        c = lax.axis_index("c")
        s = lax.axis_index("s")
        wid = s * _NC + c
        base = wid * ept
        cp_r = pltpu.async_copy(row_hbm.at[pl.ds(base, ept)], row_v, sem)
        cp_c = pltpu.async_copy(col_hbm.at[pl.ds(base, ept)], col_v, sem)
        cp_u = pltpu.async_copy(u_hbm, u_v, sem)

        zero = jnp.zeros((_L,), jnp.float32)

        def zbody(i, carry):
            v_v[pl.ds(i * _L, _L)] = zero
            return carry

        lax.fori_loop(0, nfp // _L, zbody, 0, unroll=8)
        cp_r.wait()
        cp_c.wait()
        cp_u.wait()

        def body(i, carry):
            cv = col_v[pl.ds(i * _L, _L)]
            rv = row_v[pl.ds(i * _L, _L)]
            cb = cv * fo
            rb = rv * fo
            for f in range(fo):
                gi = cb + f if f else cb
                si = rb + f if f else rb
                g = plsc.load_gather(u_v, [gi])
                plsc.addupdate_scatter(v_v, [si], g)
            return carry

        lax.fori_loop(0, ept // _L, body, 0, unroll=5)
        pltpu.sync_copy(v_v, out_hbm.at[pl.ds(wid * nfp, nfp)])

    return scatter_pass


def _combine(partials, nfp):
    return pl.pallas_call(
        _combine_body,
        grid=(_NW,),
        in_specs=[pl.BlockSpec((nfp,), lambda i: (i,))],
        out_specs=pl.BlockSpec((nfp,), lambda i: (0,)),
        out_shape=jax.ShapeDtypeStruct((nfp,), jnp.float32),
    )(partials)


def kernel(x, edge_index, W1, b1, gw1, W2, b2, gw2, W3, b3):
    n, _ = x.shape
    e = edge_index.shape[1]
    fo = W3.shape[1]
    nf = n * fo
    nfp = -(-nf // 1024) * 1024

    row = edge_index[0]
    col = edge_index[1]
    b1r = b1.reshape(1, -1)

    u2d = pl.pallas_call(
        _project_body,
        out_shape=jax.ShapeDtypeStruct((n, fo), jnp.float32),
    )(x, W1, b1r, gw1, W2, gw2, W3)
    u = u2d.reshape(nf)

    scatter = _make_scatter_pass(n, e, fo)

    p_part = scatter(u, row, col)
    v = _combine(p_part, nfp)[:nf]
    c2 = (b2 * gw2[0]) @ W3
    v = v + jnp.tile(c2, n)

    q_part = scatter(v, row, col)
    out_flat = _combine(q_part, nfp)[:nf]
    return out_flat.reshape(n, fo) + b3[None, :]


# speedup vs baseline: 23.8354x; 1.1878x over previous
import functools

import jax
import jax.numpy as jnp
from jax import lax
from jax.experimental import pallas as pl
from jax.experimental.pallas import tpu as pltpu
from jax.experimental.pallas import tpu_sc as plsc

_NC = 2
_NS = 16
_NW = _NC * _NS
_L = 16


def _project_body(x_ref, w1_ref, b1r_ref, gw1_ref, w2_ref, gw2_ref, w3_ref,
                  u_ref):
    g = (w1_ref[...] * gw1_ref[...]) @ w2_ref[...]
    wc = (g * gw2_ref[...]) @ w3_ref[...]
    c1 = (((b1r_ref[...] * gw1_ref[...]) @ w2_ref[...])
          * gw2_ref[...]) @ w3_ref[...]
    u_ref[...] = x_ref[...] @ wc + c1


def _combine_body(p_ref, o_ref):
    i = pl.program_id(0)

    @pl.when(i == 0)
    def _():
        o_ref[...] = jnp.zeros_like(o_ref)

    o_ref[...] += p_ref[...]


@functools.lru_cache(maxsize=None)
def _make_scatter_pass(n, e, fo):
    nf = n * fo
    nfp = -(-nf // 1024) * 1024
    ept = e // _NW
    assert ept * _NW == e and ept % _L == 0 and nf % _L == 0

    mesh = plsc.VectorSubcoreMesh(core_axis_name="c", subcore_axis_name="s")

    @functools.partial(
        pl.kernel,
        out_type=jax.ShapeDtypeStruct((_NW * nfp,), jnp.float32),
        mesh=mesh,
        compiler_params=pltpu.CompilerParams(needs_layout_passes=False),
        scratch_types=[
            pltpu.VMEM((nf,), jnp.float32),
            pltpu.VMEM((nfp,), jnp.float32),
            pltpu.VMEM((ept,), jnp.int32),
            pltpu.VMEM((ept,), jnp.int32),
            pltpu.SemaphoreType.DMA,
        ],
    )
    def scatter_pass(u_hbm, row_hbm, col_hbm, out_hbm,
                     u_v, v_v, row_v, col_v, sem):
        c = lax.axis_index("c")
        s = lax.axis_index("s")
        wid = s * _NC + c
        base = wid * ept
        cp_r = pltpu.async_copy(row_hbm.at[pl.ds(base, ept)], row_v, sem)
        cp_c = pltpu.async_copy(col_hbm.at[pl.ds(base, ept)], col_v, sem)
        cp_u = pltpu.async_copy(u_hbm, u_v, sem)

        zero = jnp.zeros((_L,), jnp.float32)

        @plsc.parallel_loop(0, nfp // _L, unroll=8)
        def _(i):
            v_v[pl.ds(i * _L, _L)] = zero

        cp_r.wait()
        cp_c.wait()
        cp_u.wait()

        @plsc.parallel_loop(0, ept // _L, unroll=4)
        def _(i):
            cv = col_v[pl.ds(i * _L, _L)]
            rv = row_v[pl.ds(i * _L, _L)]
            cb = cv * fo
            rb = rv * fo
            for f in range(fo):
                gi = cb + f if f else cb
                si = rb + f if f else rb
                g = plsc.load_gather(u_v, [gi])
                plsc.addupdate_scatter(v_v, [si], g)
        pltpu.sync_copy(v_v, out_hbm.at[pl.ds(wid * nfp, nfp)])

    return scatter_pass


def _combine(partials, nfp):
    return pl.pallas_call(
        _combine_body,
        grid=(_NW,),
        in_specs=[pl.BlockSpec((nfp,), lambda i: (i,))],
        out_specs=pl.BlockSpec((nfp,), lambda i: (0,)),
        out_shape=jax.ShapeDtypeStruct((nfp,), jnp.float32),
    )(partials)


def kernel(x, edge_index, W1, b1, gw1, W2, b2, gw2, W3, b3):
    n, _ = x.shape
    e = edge_index.shape[1]
    fo = W3.shape[1]
    nf = n * fo
    nfp = -(-nf // 1024) * 1024

    row = edge_index[0]
    col = edge_index[1]
    b1r = b1.reshape(1, -1)

    u2d = pl.pallas_call(
        _project_body,
        out_shape=jax.ShapeDtypeStruct((n, fo), jnp.float32),
    )(x, W1, b1r, gw1, W2, gw2, W3)
    u = u2d.reshape(nf)

    scatter = _make_scatter_pass(n, e, fo)

    p_part = scatter(u, row, col)
    v = _combine(p_part, nfp)[:nf]
    c2 = (b2 * gw2[0]) @ W3
    v = v + jnp.tile(c2, n)

    q_part = scatter(v, row, col)
    out_flat = _combine(q_part, nfp)[:nf]
    return out_flat.reshape(n, fo) + b3[None, :]


# speedup vs baseline: 28.3379x; 1.1889x over previous
import functools

import jax
import jax.numpy as jnp
from jax import lax
from jax.experimental import pallas as pl
from jax.experimental.pallas import tpu as pltpu
from jax.experimental.pallas import tpu_sc as plsc

_NC = 2
_NS = 16
_NW = _NC * _NS
_L = 16


def _project_body(x_ref, w1_ref, b1r_ref, gw1_ref, w2_ref, gw2_ref, w3_ref,
                  u_ref):
    g = (w1_ref[...] * gw1_ref[...]) @ w2_ref[...]
    wc = (g * gw2_ref[...]) @ w3_ref[...]
    c1 = (((b1r_ref[...] * gw1_ref[...]) @ w2_ref[...])
          * gw2_ref[...]) @ w3_ref[...]
    u_ref[...] = x_ref[...] @ wc + c1


def _combine_body(p_ref, o_ref, *, nfp, group):
    i = pl.program_id(0)
    acc = p_ref[pl.ds(0, nfp)]
    for g in range(1, group):
        acc += p_ref[pl.ds(g * nfp, nfp)]

    @pl.when(i == 0)
    def _():
        o_ref[...] = jnp.zeros_like(o_ref)

    o_ref[...] += acc


@functools.lru_cache(maxsize=None)
def _make_scatter_pass(n, e, fo):
    nf = n * fo
    nfp = -(-nf // 1024) * 1024
    ept = e // _NW
    assert ept * _NW == e and ept % _L == 0 and nf % _L == 0

    mesh = plsc.VectorSubcoreMesh(core_axis_name="c", subcore_axis_name="s")

    @functools.partial(
        pl.kernel,
        out_type=jax.ShapeDtypeStruct((_NW * nfp,), jnp.float32),
        mesh=mesh,
        compiler_params=pltpu.CompilerParams(needs_layout_passes=False),
        scratch_types=[
            pltpu.VMEM((nf,), jnp.float32),
            pltpu.VMEM((nfp,), jnp.float32),
            pltpu.VMEM((ept,), jnp.int32),
            pltpu.VMEM((ept,), jnp.int32),
            pltpu.SemaphoreType.DMA,
        ],
    )
    def scatter_pass(u_hbm, row_hbm, col_hbm, out_hbm,
                     u_v, v_v, row_v, col_v, sem):
        c = lax.axis_index("c")
        s = lax.axis_index("s")
        wid = s * _NC + c
        base = wid * ept
        cp_r = pltpu.async_copy(row_hbm.at[pl.ds(base, ept)], row_v, sem)
        cp_c = pltpu.async_copy(col_hbm.at[pl.ds(base, ept)], col_v, sem)
        cp_u = pltpu.async_copy(u_hbm, u_v, sem)

        zero = jnp.zeros((_L,), jnp.float32)

        @plsc.parallel_loop(0, nfp // _L, unroll=8)
        def _(i):
            v_v[pl.ds(i * _L, _L)] = zero

        cp_r.wait()
        cp_c.wait()
        cp_u.wait()

        @plsc.parallel_loop(0, ept // _L, unroll=4)
        def _(i):
            cv = col_v[pl.ds(i * _L, _L)]
            rv = row_v[pl.ds(i * _L, _L)]
            cb = cv * fo
            rb = rv * fo
            for f in range(fo):
                gi = cb + f if f else cb
                si = rb + f if f else rb
                g = plsc.load_gather(u_v, [gi])
                plsc.addupdate_scatter(v_v, [si], g)
        pltpu.sync_copy(v_v, out_hbm.at[pl.ds(wid * nfp, nfp)])

    return scatter_pass


def _combine(partials, nfp, group=4):
    return pl.pallas_call(
        functools.partial(_combine_body, nfp=nfp, group=group),
        grid=(_NW // group,),
        in_specs=[pl.BlockSpec((group * nfp,), lambda i: (i,))],
        out_specs=pl.BlockSpec((nfp,), lambda i: (0,)),
        out_shape=jax.ShapeDtypeStruct((nfp,), jnp.float32),
    )(partials)


def kernel(x, edge_index, W1, b1, gw1, W2, b2, gw2, W3, b3):
    n, _ = x.shape
    e = edge_index.shape[1]
    fo = W3.shape[1]
    nf = n * fo
    nfp = -(-nf // 1024) * 1024

    row = edge_index[0]
    col = edge_index[1]
    b1r = b1.reshape(1, -1)

    u2d = pl.pallas_call(
        _project_body,
        out_shape=jax.ShapeDtypeStruct((n, fo), jnp.float32),
    )(x, W1, b1r, gw1, W2, gw2, W3)
    u = u2d.reshape(nf)

    scatter = _make_scatter_pass(n, e, fo)

    p_part = scatter(u, row, col)
    v = _combine(p_part, nfp)[:nf]
    c2 = (b2 * gw2[0]) @ W3
    v = v + jnp.tile(c2, n)

    q_part = scatter(v, row, col)
    out_flat = _combine(q_part, nfp)[:nf]
    return out_flat.reshape(n, fo) + b3[None, :]
